# baseline (speedup 1.0000x reference)
import functools

import jax
import jax.numpy as jnp
from jax import lax
from jax.experimental import pallas as pl
from jax.experimental.pallas import tpu as pltpu
from jax.experimental.pallas import tpu_sc as plsc

N, M, U, D = 32, 4096, 2048, 256
CH = 128
NCH = M // CH
UCH = U // CH
R = 3
G = 2


@functools.lru_cache(maxsize=1)
def _build_sc_kernel():
    info = plsc.get_sparse_core_info()
    nc = info.num_cores
    mesh = plsc.VectorSubcoreMesh(core_axis_name="c", subcore_axis_name="s")

    @functools.partial(
        pl.kernel,
        out_type=jax.ShapeDtypeStruct((N * M, D), jnp.float32),
        mesh=mesh,
        scratch_types=[
            pltpu.VMEM((8, 128), jnp.int32),
            pltpu.VMEM((R, CH, D), jnp.float32),
            [pltpu.VMEM((CH,), jnp.int32)] * R,
            [pltpu.VMEM((CH,), jnp.int32)] * R,
            [pltpu.SemaphoreType.DMA] * R,
            [pltpu.SemaphoreType.DMA] * R,
        ],
    )
    def k(var_hbm, upd_hbm, idx_hbm, out_hbm, idx_v, buf, sidx, didx, sg, ss):
        wid = lax.axis_index("s") * nc + lax.axis_index("c")
        pltpu.sync_copy(idx_hbm.at[wid], idx_v)
        v = idx_v[0, pl.ds(0, 16)]
        off = v[0]
        ln = v[1]
        end = off + ln
        vbase = wid * M
        ubase = wid * U

        def al(x):
            return pl.multiple_of(x, 8)

        def copy_cond(kk):
            b = kk * CH
            return jnp.logical_not(jnp.logical_and(off <= b, b + CH <= end))

        def g_var(kk):
            p = kk % R
            return pltpu.make_async_copy(
                var_hbm.at[pl.ds(al(vbase + kk * CH), CH)], buf.at[p], sg[p]
            )

        def s_out(kk):
            p = kk % R
            return pltpu.make_async_copy(
                buf.at[p], out_hbm.at[pl.ds(al(vbase + kk * CH), CH)], ss[p]
            )

        for kk in range(NCH + G):
            if kk < NCH:
                if kk >= R:

                    @pl.when(copy_cond(kk - R))
                    def _():
                        s_out(kk - R).wait()

                @pl.when(copy_cond(kk))
                def _():
                    g_var(kk).start()

            if kk >= G:
                j = kk - G

                @pl.when(copy_cond(j))
                def _():
                    g_var(j).wait()
                    s_out(j).start()

        for j in range(NCH - R, NCH):

            @pl.when(copy_cond(j))
            def _():
                s_out(j).wait()

        nch = (ln + CH - 1) // CH
        iota = lax.iota(jnp.int32, 16)

        def g_upd(t):
            p = t % R
            return pltpu.make_async_copy(upd_hbm.at[sidx[p]], buf.at[p], sg[p])

        def s_upd(t):
            p = t % R
            return pltpu.make_async_copy(buf.at[p], out_hbm.at[didx[p]], ss[p])

        for t in range(UCH + G):
            if t < UCH:
                if t >= R:

                    @pl.when(t - R < nch)
                    def _():
                        s_upd(t - R).wait()

                @pl.when(t < nch)
                def _():
                    p = t % R
                    for b in range(CH // 16):
                        q = jnp.minimum(t * CH + b * 16 + iota, ln - 1)
                        sidx[p][pl.ds(b * 16, 16)] = ubase + q
                        didx[p][pl.ds(b * 16, 16)] = vbase + off + q
                    g_upd(t).start()

            if t >= G:
                j = t - G

                @pl.when(j < nch)
                def _():
                    g_upd(j).wait()
                    s_upd(j).start()

        for j in range(UCH - R, UCH):

            @pl.when(j < nch)
            def _():
                s_upd(j).wait()

    return k


def kernel(varRef, indice, updates, mask, reduce, axis):
    idx = indice.astype(jnp.int32)
    off = jnp.clip(idx[:, 0], 0, M)
    ln = jnp.clip(idx[:, 1], 0, M - off)
    ln = jnp.where(mask, ln, 0)
    idx3 = jnp.zeros((N, 8, 128), jnp.int32)
    idx3 = idx3.at[:, 0, 0].set(off).at[:, 0, 1].set(ln)
    out = _build_sc_kernel()(
        varRef.reshape(N * M, D), updates.reshape(N * U, D), idx3
    )
    return out.reshape(N, M, D)


# speedup vs baseline: 1.1190x; 1.1190x over previous
import functools

import jax
import jax.numpy as jnp
from jax import lax
from jax.experimental import pallas as pl
from jax.experimental.pallas import tpu as pltpu
from jax.experimental.pallas import tpu_sc as plsc

N, M, U, D = 32, 4096, 2048, 256
CH = 32
NCH = M // CH
UCH = U // CH
R = 12
G = 6


@functools.lru_cache(maxsize=1)
def _build_sc_kernel():
    info = plsc.get_sparse_core_info()
    nc = info.num_cores
    mesh = plsc.VectorSubcoreMesh(core_axis_name="c", subcore_axis_name="s")

    @functools.partial(
        pl.kernel,
        out_type=jax.ShapeDtypeStruct((N * M, D), jnp.float32),
        mesh=mesh,
        scratch_types=[
            pltpu.VMEM((8, 128), jnp.int32),
            pltpu.VMEM((R, CH, D), jnp.float32),
            [pltpu.VMEM((CH,), jnp.int32)] * R,
            [pltpu.VMEM((CH,), jnp.int32)] * R,
            [pltpu.SemaphoreType.DMA] * R,
            [pltpu.SemaphoreType.DMA] * R,
        ],
    )
    def k(var_hbm, upd_hbm, idx_hbm, out_hbm, idx_v, buf, sidx, didx, sg, ss):
        wid = lax.axis_index("s") * nc + lax.axis_index("c")
        pltpu.sync_copy(idx_hbm.at[wid], idx_v)
        v = idx_v[0, pl.ds(0, 16)]
        off = v[0]
        ln = v[1]
        end = off + ln
        vbase = wid * M
        ubase = wid * U

        def al(x):
            return pl.multiple_of(x, 8)

        def copy_cond(kk):
            b = kk * CH
            return jnp.logical_not(jnp.logical_and(off <= b, b + CH <= end))

        def g_var(kk):
            p = kk % R
            return pltpu.make_async_copy(
                var_hbm.at[pl.ds(al(vbase + kk * CH), CH)], buf.at[p], sg[p]
            )

        def s_out(kk):
            p = kk % R
            return pltpu.make_async_copy(
                buf.at[p], out_hbm.at[pl.ds(al(vbase + kk * CH), CH)], ss[p]
            )

        for kk in range(NCH + G):
            if kk < NCH:
                if kk >= R:

                    @pl.when(copy_cond(kk - R))
                    def _():
                        s_out(kk - R).wait()

                @pl.when(copy_cond(kk))
                def _():
                    g_var(kk).start()

            if kk >= G:
                j = kk - G

                @pl.when(copy_cond(j))
                def _():
                    g_var(j).wait()
                    s_out(j).start()

        for j in range(NCH - R, NCH):

            @pl.when(copy_cond(j))
            def _():
                s_out(j).wait()

        nch = (ln + CH - 1) // CH
        iota = lax.iota(jnp.int32, 16)

        def g_upd(t):
            p = t % R
            return pltpu.make_async_copy(upd_hbm.at[sidx[p]], buf.at[p], sg[p])

        def s_upd(t):
            p = t % R
            return pltpu.make_async_copy(buf.at[p], out_hbm.at[didx[p]], ss[p])

        for t in range(UCH + G):
            if t < UCH:
                if t >= R:

                    @pl.when(t - R < nch)
                    def _():
                        s_upd(t - R).wait()

                @pl.when(t < nch)
                def _():
                    p = t % R
                    for b in range(CH // 16):
                        q = jnp.minimum(t * CH + b * 16 + iota, ln - 1)
                        sidx[p][pl.ds(b * 16, 16)] = ubase + q
                        didx[p][pl.ds(b * 16, 16)] = vbase + off + q
                    g_upd(t).start()

            if t >= G:
                j = t - G

                @pl.when(j < nch)
                def _():
                    g_upd(j).wait()
                    s_upd(j).start()

        for j in range(UCH - R, UCH):

            @pl.when(j < nch)
            def _():
                s_upd(j).wait()

    return k


def kernel(varRef, indice, updates, mask, reduce, axis):
    idx = indice.astype(jnp.int32)
    off = jnp.clip(idx[:, 0], 0, M)
    ln = jnp.clip(idx[:, 1], 0, M - off)
    ln = jnp.where(mask, ln, 0)
    idx3 = jnp.zeros((N, 8, 128), jnp.int32)
    idx3 = idx3.at[:, 0, 0].set(off).at[:, 0, 1].set(ln)
    out = _build_sc_kernel()(
        varRef.reshape(N * M, D), updates.reshape(N * U, D), idx3
    )
    return out.reshape(N, M, D)
